# baseline (speedup 1.0000x reference)
import math

import jax
import jax.numpy as jnp
import numpy as np
from jax.experimental import pallas as pl

_NBINS = 9
_GW = 16
_SLAB = 16
_FETCH = _SLAB + 8


def _gauss_window(h: int, w: int) -> np.ndarray:
    n = np.arange(_GW, dtype=np.float32)
    n = (n - n.mean()) / (_GW // 2)
    g1 = np.exp(-0.5 * n * n)
    g2 = np.outer(g1, g1).astype(np.float32)
    g2 = g2 / g2.sum()
    return np.tile(g2, (h // _GW, w // _GW))


def _hog_program(xp_ref, gk_ref, o_ref):
    xp = xp_ref[0, 0:_SLAB + 2, :].astype(jnp.float32)
    gk = gk_ref[...]
    h = _SLAB
    w = xp.shape[1] - 2

    v = xp[0:h, :] + 2.0 * xp[1:h + 1, :] + xp[2:h + 2, :]
    gx = v[:, 0:w] - v[:, 2:w + 2]
    hz = xp[:, 0:w] + 2.0 * xp[:, 1:w + 1] + xp[:, 2:w + 2]
    gy = hz[0:h, :] - hz[2:h + 2, :]

    norm = jnp.sqrt(gx * gx + gy * gy) * gk

    pos = (gx > 0.0) | ((gx == 0.0) & (gy > 0.0))
    s = jnp.where(pos, 1.0, -1.0)
    gxc = gx * s
    gyc = gy * s

    b = []
    for m in range(1, _NBINS):
        cm = math.cos(m * math.pi / _NBINS)
        sm = math.sin(m * math.pi / _NBINS)
        b.append(gxc * cm - gyc * sm >= 0.0)

    zero = jnp.zeros_like(norm)
    o_ref[0] = jnp.where(b[0], zero, norm)
    for k in range(1, _NBINS - 1):
        o_ref[k] = jnp.where(b[k - 1] & ~b[k], norm, zero)
    o_ref[_NBINS - 1] = jnp.where(b[_NBINS - 2], norm, zero)


def _hog_call(xp, gk, h, interpret=False):
    n, hpad, wp = xp.shape
    w = wp - 2
    nslab = h // _SLAB
    return pl.pallas_call(
        _hog_program,
        grid=(n, nslab),
        in_specs=[
            pl.BlockSpec(
                (pl.Element(1), pl.Element(_FETCH), pl.Element(wp)),
                lambda i, j: (i, _SLAB * j, 0),
            ),
            pl.BlockSpec((_SLAB, w), lambda i, j: (j, 0)),
        ],
        out_specs=pl.BlockSpec(
            (pl.Squeezed(), _NBINS, _SLAB, w), lambda i, j: (i, 0, j, 0)
        ),
        out_shape=jax.ShapeDtypeStruct((n, _NBINS, h, w), jnp.float32),
        interpret=interpret,
    )(xp, gk)


def kernel(x):
    bsz, c, h, w = x.shape
    xr = x.reshape(bsz * c, h, w).astype(jnp.bfloat16)
    xp = jnp.pad(xr, ((0, 0), (1, _FETCH - _SLAB - 1), (1, 1)), mode="reflect")
    gk = jnp.asarray(_gauss_window(h, w))
    out = _hog_call(xp, gk, h)
    return out.reshape(bsz, c, _NBINS, h, w)


# speedup vs baseline: 2.5134x; 2.5134x over previous
import math

import jax
import jax.numpy as jnp
import numpy as np
from jax.experimental import pallas as pl

_NBINS = 9
_GW = 16
_SLAB = 8


def _gauss_window(h: int, w: int) -> np.ndarray:
    n = np.arange(_GW, dtype=np.float32)
    n = (n - n.mean()) / (_GW // 2)
    g1 = np.exp(-0.5 * n * n)
    g2 = np.outer(g1, g1).astype(np.float32)
    g2 = g2 / g2.sum()
    return np.tile(g2, (h // _GW, w // _GW))


def _hog_program(xp_ref, gk_ref, o_ref):
    hp = xp_ref.shape[1]
    wp = xp_ref.shape[2]
    h, w = hp - 2, wp - 2

    def slab(t, carry):
        r0 = _SLAB * t
        xp = xp_ref[0, pl.ds(r0, _SLAB + 2), :].astype(jnp.float32)
        gk = gk_ref[pl.ds(r0, _SLAB), :]

        v = xp[0:_SLAB, :] + 2.0 * xp[1:_SLAB + 1, :] + xp[2:_SLAB + 2, :]
        gx = v[:, 0:w] - v[:, 2:w + 2]
        hz = xp[:, 0:w] + 2.0 * xp[:, 1:w + 1] + xp[:, 2:w + 2]
        gy = hz[0:_SLAB, :] - hz[2:_SLAB + 2, :]

        norm = jnp.sqrt(gx * gx + gy * gy) * gk

        pos = (gx > 0.0) | ((gx == 0.0) & (gy > 0.0))
        s = jnp.where(pos, 1.0, -1.0)
        gxc = gx * s
        gyc = gy * s

        b = []
        for m in range(1, _NBINS):
            cm = math.cos(m * math.pi / _NBINS)
            sm = math.sin(m * math.pi / _NBINS)
            b.append(gxc * cm - gyc * sm >= 0.0)

        zero = jnp.zeros_like(norm)
        o_ref[0, 0, pl.ds(r0, _SLAB), :] = jnp.where(b[0], zero, norm)
        for k in range(1, _NBINS - 1):
            o_ref[0, k, pl.ds(r0, _SLAB), :] = jnp.where(b[k - 1] & ~b[k], norm, zero)
        o_ref[0, _NBINS - 1, pl.ds(r0, _SLAB), :] = jnp.where(b[_NBINS - 2], norm, zero)
        return carry

    jax.lax.fori_loop(0, h // _SLAB, slab, 0)


def _hog_call(xp, gk, interpret=False):
    n, hp, wp = xp.shape
    h, w = hp - 2, wp - 2
    return pl.pallas_call(
        _hog_program,
        grid=(n,),
        in_specs=[
            pl.BlockSpec((1, hp, wp), lambda i: (i, 0, 0)),
            pl.BlockSpec((h, w), lambda i: (0, 0)),
        ],
        out_specs=pl.BlockSpec((1, _NBINS, h, w), lambda i: (i, 0, 0, 0)),
        out_shape=jax.ShapeDtypeStruct((n, _NBINS, h, w), jnp.float32),
        interpret=interpret,
    )(xp, gk)


def kernel(x):
    bsz, c, h, w = x.shape
    xr = x.reshape(bsz * c, h, w).astype(jnp.bfloat16)
    xp = jnp.pad(xr, ((0, 0), (1, 1), (1, 1)), mode="reflect")
    gk = jnp.asarray(_gauss_window(h, w))
    out = _hog_call(xp, gk)
    return out.reshape(bsz, c, _NBINS, h, w)


# speedup vs baseline: 5.1965x; 2.0675x over previous
import math

import jax
import jax.numpy as jnp
import numpy as np
from jax.experimental import pallas as pl

_NBINS = 9
_GW = 16
_SLAB = 8


def _gauss_window(h: int, w: int) -> np.ndarray:
    n = np.arange(_GW, dtype=np.float32)
    n = (n - n.mean()) / (_GW // 2)
    g1 = np.exp(-0.5 * n * n)
    g2 = np.outer(g1, g1).astype(np.float32)
    g2 = g2 / g2.sum()
    return np.tile(g2, (h // _GW, w // _GW))


def _hog_program(xp_ref, gk_ref, o_ref):
    xp = xp_ref[0].astype(jnp.float32)
    gk = gk_ref[...]
    h = xp.shape[0] - 2
    w = xp.shape[1] - 2

    v = xp[0:h, :] + 2.0 * xp[1:h + 1, :] + xp[2:h + 2, :]
    gx = v[:, 0:w] - v[:, 2:w + 2]
    hz = xp[:, 0:w] + 2.0 * xp[:, 1:w + 1] + xp[:, 2:w + 2]
    gy = hz[0:h, :] - hz[2:h + 2, :]

    norm = jnp.sqrt(gx * gx + gy * gy) * gk

    pos = (gx > 0.0) | ((gx == 0.0) & (gy > 0.0))
    s = jnp.where(pos, 1.0, -1.0)
    gxc = gx * s
    gyc = gy * s

    b = []
    for m in range(1, _NBINS):
        cm = math.cos(m * math.pi / _NBINS)
        sm = math.sin(m * math.pi / _NBINS)
        b.append(gxc * cm - gyc * sm >= 0.0)

    zero = jnp.zeros_like(norm)
    o_ref[0, 0] = jnp.where(b[0], zero, norm)
    for k in range(1, _NBINS - 1):
        o_ref[0, k] = jnp.where(b[k - 1] & ~b[k], norm, zero)
    o_ref[0, _NBINS - 1] = jnp.where(b[_NBINS - 2], norm, zero)


def _hog_call(xp, gk, interpret=False):
    n, hp, wp = xp.shape
    h, w = hp - 2, wp - 2
    return pl.pallas_call(
        _hog_program,
        grid=(n,),
        in_specs=[
            pl.BlockSpec((1, hp, wp), lambda i: (i, 0, 0)),
            pl.BlockSpec((h, w), lambda i: (0, 0)),
        ],
        out_specs=pl.BlockSpec((1, _NBINS, h, w), lambda i: (i, 0, 0, 0)),
        out_shape=jax.ShapeDtypeStruct((n, _NBINS, h, w), jnp.float32),
        interpret=interpret,
    )(xp, gk)


def kernel(x):
    bsz, c, h, w = x.shape
    xr = x.reshape(bsz * c, h, w).astype(jnp.bfloat16)
    xp = jnp.pad(xr, ((0, 0), (1, 1), (1, 1)), mode="reflect")
    gk = jnp.asarray(_gauss_window(h, w))
    out = _hog_call(xp, gk)
    return out.reshape(bsz, c, _NBINS, h, w)


# speedup vs baseline: 5.3968x; 1.0385x over previous
import math

import jax
import jax.numpy as jnp
import numpy as np
from jax.experimental import pallas as pl

_NBINS = 9
_GW = 16
_SLAB = 8


def _gauss_window(h: int, w: int) -> np.ndarray:
    n = np.arange(_GW, dtype=np.float32)
    n = (n - n.mean()) / (_GW // 2)
    g1 = np.exp(-0.5 * n * n)
    g2 = np.outer(g1, g1).astype(np.float32)
    g2 = g2 / g2.sum()
    return np.tile(g2, (h // _GW, w // _GW))


def _hog_program(xp_ref, gk_ref, o_ref):
    xp = xp_ref[0].astype(jnp.float32)
    gk = gk_ref[...]
    h = xp.shape[0] - 2
    w = xp.shape[1] - 2

    v = xp[0:h, :] + 2.0 * xp[1:h + 1, :] + xp[2:h + 2, :]
    gx = v[:, 0:w] - v[:, 2:w + 2]
    hz = xp[:, 0:w] + 2.0 * xp[:, 1:w + 1] + xp[:, 2:w + 2]
    gy = hz[0:h, :] - hz[2:h + 2, :]

    norm = jnp.sqrt(gx * gx + gy * gy) * gk

    u = gy / gx
    b = []
    for m in range(1, _NBINS):
        b.append(u <= math.cos(m * math.pi / _NBINS) / math.sin(m * math.pi / _NBINS))
    z = gx == 0.0

    zero = jnp.zeros_like(norm)
    o_ref[0, 0] = jnp.where(~b[0] | z, norm, zero)
    for k in range(1, _NBINS - 1):
        o_ref[0, k] = jnp.where(b[k - 1] & ~b[k], norm, zero)
    o_ref[0, _NBINS - 1] = jnp.where(b[_NBINS - 2] & ~z, norm, zero)


def _hog_call(xp, gk, interpret=False):
    n, hp, wp = xp.shape
    h, w = hp - 2, wp - 2
    return pl.pallas_call(
        _hog_program,
        grid=(n,),
        in_specs=[
            pl.BlockSpec((1, hp, wp), lambda i: (i, 0, 0)),
            pl.BlockSpec((h, w), lambda i: (0, 0)),
        ],
        out_specs=pl.BlockSpec((1, _NBINS, h, w), lambda i: (i, 0, 0, 0)),
        out_shape=jax.ShapeDtypeStruct((n, _NBINS, h, w), jnp.float32),
        interpret=interpret,
    )(xp, gk)


def kernel(x):
    bsz, c, h, w = x.shape
    xr = x.reshape(bsz * c, h, w).astype(jnp.bfloat16)
    xp = jnp.pad(xr, ((0, 0), (1, 1), (1, 1)), mode="reflect")
    gk = jnp.asarray(_gauss_window(h, w))
    out = _hog_call(xp, gk)
    return out.reshape(bsz, c, _NBINS, h, w)
